# baseline (speedup 1.0000x reference)
import functools

import jax
import jax.numpy as jnp
from jax import lax
from jax.experimental import pallas as pl
from jax.experimental.pallas import tpu as pltpu
from jax.experimental.pallas import tpu_sc as plsc

NB_USERS = 10000
NB_MOVIES = 10000
K = 128
MAX_RATING = 5.0

NC = 2
NS = 16
NW = NC * NS

CH = 128
NCH = 128
N_PAD = NW * CH * NCH
NPARTS = 2
NCH_P = NCH // NPARTS
PART = N_PAD // NPARTS
U_PAD = 10240
U_ROWS = U_PAD // NS
DUMMY_USER = NB_USERS

_ROW_CHUNK = 2048

_SC_PARAMS = pltpu.CompilerParams(use_tc_tiling_on_sc=False)
_SC_PARAMS_NL = pltpu.CompilerParams(use_tc_tiling_on_sc=False,
                                     needs_layout_passes=False)


def _mlp_body(g_ref, dr_ref, w1_ref, b1_ref, w2t_ref, b2_ref, v_ref):
    dr = dr_ref[:]
    s = jax.nn.sigmoid((dr * (1.0 / MAX_RATING)) * w1_ref[:] + b1_ref[:])
    t = g_ref[:] * s
    z = jnp.dot(t, w2t_ref[:], preferred_element_type=jnp.float32) + b2_ref[:]
    v_ref[:] = jax.nn.sigmoid(z)


def _mlp(g, dr, w1_row, b1_row, w2t, b2_row):
    n = g.shape[0]
    c = _ROW_CHUNK
    return pl.pallas_call(
        _mlp_body,
        grid=(n // c,),
        in_specs=[
            pl.BlockSpec((c, K), lambda i: (i, 0)),
            pl.BlockSpec((c, 1), lambda i: (i, 0)),
            pl.BlockSpec((1, K), lambda i: (0, 0)),
            pl.BlockSpec((1, K), lambda i: (0, 0)),
            pl.BlockSpec((K, K), lambda i: (0, 0)),
            pl.BlockSpec((1, K), lambda i: (0, 0)),
        ],
        out_specs=pl.BlockSpec((c, K), lambda i: (i, 0)),
        out_shape=jax.ShapeDtypeStruct((n, K), jnp.float32),
    )(g, dr, w1_row, b1_row, w2t, b2_row)


def _gather_body(tab_hbm, dm_hbm, g_out,
                 vb0, vb1, idx_b, sv0, sv1, sw0, sw1):
    c = lax.axis_index("c")
    s = lax.axis_index("s")
    w = s * NC + c
    rows_per_w = CH * NCH_P
    base0 = w * rows_per_w

    pltpu.sync_copy(dm_hbm.at[pl.ds(base0, rows_per_w)], idx_b)

    GRP = 2 * CH
    ng = NCH_P // 2
    slots = ((vb0, sv0, sw0), (vb1, sv1, sw1))

    def fire(vb, sv, g):
        for j in range(2):
            idx = idx_b.at[pl.ds((g * 2 + j) * CH, CH)]
            pltpu.async_copy(tab_hbm.at[idx], vb.at[pl.ds(j * CH, CH)], sv)

    for b, (vb, sv, sw) in enumerate(slots):
        fire(vb, sv, b)

    def group_step(g, carry):
        for b, (vb, sv, sw) in enumerate(slots):
            gg = g * 2 + b
            for j in range(2):
                pltpu.make_async_copy(
                    tab_hbm.at[idx_b.at[pl.ds(0, CH)]],
                    vb.at[pl.ds(j * CH, CH)], sv).wait()
            pltpu.async_copy(vb, g_out.at[pl.ds(base0 + gg * GRP, GRP)], sw)

            @pl.when(gg + 2 < ng)
            def _():
                pltpu.make_async_copy(
                    vb, g_out.at[pl.ds(base0, GRP)], sw).wait()
                fire(vb, sv, gg + 2)
        return carry

    lax.fori_loop(0, ng // 2, group_step, 0)
    for b, (vb, sv, sw) in enumerate(slots):
        pltpu.make_async_copy(vb, g_out.at[pl.ds(base0, GRP)], sw).wait()


def _gather(tab, dm_part):
    mesh = plsc.VectorSubcoreMesh(core_axis_name="c", subcore_axis_name="s")
    return pl.kernel(
        _gather_body,
        out_type=jax.ShapeDtypeStruct((PART, K), jnp.float32),
        mesh=mesh,
        compiler_params=_SC_PARAMS,
        scratch_types=[
            pltpu.VMEM((2 * CH, K), jnp.float32),
            pltpu.VMEM((2 * CH, K), jnp.float32),
            pltpu.VMEM((CH * NCH_P,), jnp.int32),
            pltpu.SemaphoreType.DMA,
            pltpu.SemaphoreType.DMA,
            pltpu.SemaphoreType.DMA,
            pltpu.SemaphoreType.DMA,
        ],
    )(tab, dm_part)


def _scatter_body(v0_hbm, v1_hbm, du_hbm, acc_out,
                  vb0, vb1, ib0, ib1, zb, acc_sh, sv0, sv1, si0, si1):
    c = lax.axis_index("c")
    s = lax.axis_index("s")
    w = s * NC + c

    zeros16 = jnp.zeros((16,), jnp.float32)

    def fill_zb(i, carry):
        zb[i // 8, pl.ds((i % 8) * 16, 16)] = zeros16
        return carry
    lax.fori_loop(0, 8 * zb.shape[0], fill_zb, 0)

    row0 = s * U_ROWS
    for k in range(U_ROWS // zb.shape[0]):
        pltpu.sync_copy(zb, acc_sh.at[pl.ds(row0 + k * zb.shape[0], zb.shape[0])])
    plsc.subcore_barrier()

    slots = ((vb0, ib0, sv0, si0), (vb1, ib1, sv1, si1))

    for p, v_hbm in enumerate((v0_hbm, v1_hbm)):
        vbase = w * (CH * NCH_P)
        dbase = p * PART + vbase
        for b, (vb, ib, sv, si) in enumerate(slots):
            pltpu.async_copy(v_hbm.at[pl.ds(vbase + b * CH, CH)], vb, sv)
            pltpu.async_copy(du_hbm.at[pl.ds(dbase + b * CH, CH)], ib, si)

        def chunk_pair(jj, carry):
            for b, (vb, ib, sv, si) in enumerate(slots):
                g = jj * 2 + b
                pltpu.make_async_copy(v_hbm.at[pl.ds(0, CH)], vb, sv).wait()
                pltpu.make_async_copy(du_hbm.at[pl.ds(0, CH)], ib, si).wait()
                pltpu.sync_copy(vb, acc_sh.at[ib], add=True)

                @pl.when(g + 2 < NCH_P)
                def _():
                    pltpu.async_copy(
                        v_hbm.at[pl.ds(vbase + (g + 2) * CH, CH)], vb, sv)
                    pltpu.async_copy(
                        du_hbm.at[pl.ds(dbase + (g + 2) * CH, CH)], ib, si)
            return carry

        lax.fori_loop(0, NCH_P // 2, chunk_pair, 0)

    plsc.subcore_barrier()
    pltpu.sync_copy(acc_sh.at[pl.ds(row0, U_ROWS)],
                    acc_out.at[c, pl.ds(row0, U_ROWS)])


def _scatter(v0, v1, du):
    mesh = plsc.VectorSubcoreMesh(core_axis_name="c", subcore_axis_name="s")
    return pl.kernel(
        _scatter_body,
        out_type=jax.ShapeDtypeStruct((NC, U_PAD, K), jnp.float32),
        mesh=mesh,
        compiler_params=_SC_PARAMS,
        scratch_types=[
            pltpu.VMEM((CH, K), jnp.float32),
            pltpu.VMEM((CH, K), jnp.float32),
            pltpu.VMEM((CH,), jnp.int32),
            pltpu.VMEM((CH,), jnp.int32),
            pltpu.VMEM((80, K), jnp.float32),
            pltpu.VMEM_SHARED((U_PAD, K), jnp.float32),
            pltpu.SemaphoreType.DMA,
            pltpu.SemaphoreType.DMA,
            pltpu.SemaphoreType.DMA,
            pltpu.SemaphoreType.DMA,
        ],
    )(v0, v1, du)


def _count_body(du_hbm, cnt_out, ib0, ib1, ones_b, zc, cnt_sh, si0, si1):
    c = lax.axis_index("c")
    s = lax.axis_index("s")
    w = s * NC + c

    zeros16 = jnp.zeros((16,), jnp.float32)
    ones16 = jnp.ones((16,), jnp.float32)

    def fill_zc(i, carry):
        zc[i, :] = zeros16
        return carry
    lax.fori_loop(0, zc.shape[0], fill_zc, 0)

    def fill_ones(i, carry):
        ones_b[i, :] = ones16
        return carry
    lax.fori_loop(0, ones_b.shape[0], fill_ones, 0)

    row0 = s * U_ROWS
    pltpu.sync_copy(zc, cnt_sh.at[pl.ds(row0, U_ROWS)])
    plsc.subcore_barrier()

    base0 = w * (CH * NCH)
    slots = ((ib0, si0), (ib1, si1))
    for b, (ib, si) in enumerate(slots):
        pltpu.async_copy(du_hbm.at[pl.ds(base0 + b * CH, CH)], ib, si)

    def chunk_pair(jj, carry):
        for b, (ib, si) in enumerate(slots):
            g = jj * 2 + b
            pltpu.make_async_copy(du_hbm.at[pl.ds(0, CH)], ib, si).wait()
            pltpu.sync_copy(ones_b, cnt_sh.at[ib], add=True)

            @pl.when(g + 2 < NCH)
            def _():
                nb = base0 + (g + 2) * CH
                pltpu.async_copy(du_hbm.at[pl.ds(nb, CH)], ib, si)
        return carry

    lax.fori_loop(0, NCH // 2, chunk_pair, 0)
    plsc.subcore_barrier()

    pltpu.sync_copy(cnt_sh.at[pl.ds(row0, U_ROWS)],
                    cnt_out.at[c, pl.ds(row0, U_ROWS)])


def _count(du):
    mesh = plsc.VectorSubcoreMesh(core_axis_name="c", subcore_axis_name="s")
    return pl.kernel(
        _count_body,
        out_type=jax.ShapeDtypeStruct((NC, U_PAD, 16), jnp.float32),
        mesh=mesh,
        compiler_params=_SC_PARAMS,
        scratch_types=[
            pltpu.VMEM((CH,), jnp.int32),
            pltpu.VMEM((CH,), jnp.int32),
            pltpu.VMEM((CH, 16), jnp.float32),
            pltpu.VMEM((U_ROWS, 16), jnp.float32),
            pltpu.VMEM_SHARED((U_PAD, 16), jnp.float32),
            pltpu.SemaphoreType.DMA,
            pltpu.SemaphoreType.DMA,
        ],
    )(du)


E_TILE = 1664
E_CHN = 13
E_PAD = NW * E_TILE


def _eval_body(uf_hbm, mov_hbm, xu_hbm, xm_hbm, out_hbm,
               ub0, mb0, ub1, mb1, xui, xmi, ob, su0, sm0, su1, sm1):
    c = lax.axis_index("c")
    s = lax.axis_index("s")
    w = s * NC + c
    base0 = w * E_TILE

    pltpu.sync_copy(xu_hbm.at[pl.ds(base0, E_TILE)], xui)
    pltpu.sync_copy(xm_hbm.at[pl.ds(base0, E_TILE)], xmi)

    slots = ((ub0, mb0, su0, sm0), (ub1, mb1, su1, sm1))

    def fire(b, ch):
        ub, mb, su, sm = slots[b]
        pltpu.async_copy(uf_hbm.at[xui.at[pl.ds(ch * CH, CH)]], ub, su)
        pltpu.async_copy(mov_hbm.at[xmi.at[pl.ds(ch * CH, CH)]], mb, sm)

    for b in range(2):
        fire(b, b)

    iota16 = jnp.arange(16, dtype=jnp.int32)

    def chunk_step(ch, carry):
        for b in range(2):
            @pl.when(ch * 2 + b < E_CHN)
            def _():
                cc = ch * 2 + b
                ub, mb, su, sm = slots[b]
                pltpu.make_async_copy(
                    uf_hbm.at[xui.at[pl.ds(0, CH)]], ub, su).wait()
                pltpu.make_async_copy(
                    mov_hbm.at[xmi.at[pl.ds(0, CH)]], mb, sm).wait()

                def group(g16, carry2):
                    rows16 = g16 * 16 + iota16

                    def jblk(jj, acc):
                        for j2 in range(32):
                            cols = jnp.full((16,), 0, jnp.int32) + (jj * 32 + j2)
                            u = plsc.load_gather(ub, [rows16, cols])
                            m = plsc.load_gather(mb, [rows16, cols])
                            acc = acc + u * m
                        return acc

                    acc = lax.fori_loop(0, 4, jblk,
                                        jnp.zeros((16,), jnp.float32))
                    ob[pl.ds(cc * CH + g16 * 16, 16)] = acc
                    return carry2

                lax.fori_loop(0, CH // 16, group, 0)

                @pl.when(cc + 2 < E_CHN)
                def _():
                    fire(b, cc + 2)
        return carry

    lax.fori_loop(0, (E_CHN + 1) // 2, chunk_step, 0)
    pltpu.sync_copy(ob, out_hbm.at[pl.ds(base0, E_TILE)])


def _eval_dots(uf, mov, xu, xm):
    mesh = plsc.VectorSubcoreMesh(core_axis_name="c", subcore_axis_name="s")
    return pl.kernel(
        _eval_body,
        out_type=jax.ShapeDtypeStruct((E_PAD,), jnp.float32),
        mesh=mesh,
        compiler_params=_SC_PARAMS_NL,
        scratch_types=[
            pltpu.VMEM((CH, K), jnp.float32),
            pltpu.VMEM((CH, K), jnp.float32),
            pltpu.VMEM((CH, K), jnp.float32),
            pltpu.VMEM((CH, K), jnp.float32),
            pltpu.VMEM((E_TILE,), jnp.int32),
            pltpu.VMEM((E_TILE,), jnp.int32),
            pltpu.VMEM((E_TILE,), jnp.float32),
            pltpu.SemaphoreType.DMA,
            pltpu.SemaphoreType.DMA,
            pltpu.SemaphoreType.DMA,
            pltpu.SemaphoreType.DMA,
        ],
    )(uf, mov, xu, xm)


def kernel(eval_xs, data_x, data_ratings, ignore_if_seen, movies_features,
           W1, b1, W2, b2, Wr, br):
    x_users = eval_xs[:, 0]
    x_movies = eval_xs[:, 1]
    n = data_x.shape[0]
    pad = N_PAD - n
    du = jnp.concatenate([data_x[:, 0],
                          jnp.full((pad,), DUMMY_USER, jnp.int32)])
    dm = jnp.concatenate([data_x[:, 1], jnp.zeros((pad,), jnp.int32)])
    dr = jnp.concatenate([data_ratings, jnp.zeros((pad,), jnp.float32)])

    w1r = W1.reshape(1, K)
    b1r = b1.reshape(1, K)
    w2t = W2.T
    b2r = b2.reshape(1, K)

    cnt_p = _count(du)

    vs = {}
    for p in (0, 1):
        gp = _gather(movies_features, dm[p * PART:(p + 1) * PART])
        vp = _mlp(gp, dr[p * PART:(p + 1) * PART, None], w1r, b1r, w2t, b2r)
        vs[p] = vp

    acc_p = _scatter(vs[0], vs[1], du)
    acc = acc_p[0] + acc_p[1]
    cnt = cnt_p[0, :, 0] + cnt_p[1, :, 0]

    scale = Wr[0, 0] * (MAX_RATING / K)
    uf_scaled = acc * (scale / cnt[:, None])

    epad = E_PAD - eval_xs.shape[0]
    xu = jnp.concatenate([x_users, jnp.zeros((epad,), jnp.int32)])
    xm = jnp.concatenate([x_movies, jnp.zeros((epad,), jnp.int32)])
    dots = _eval_dots(uf_scaled, movies_features, xu, xm)
    return dots[: eval_xs.shape[0]] + br[0] * MAX_RATING


# speedup vs baseline: 1.4938x; 1.4938x over previous
import functools

import jax
import jax.numpy as jnp
from jax import lax
from jax.experimental import pallas as pl
from jax.experimental.pallas import tpu as pltpu
from jax.experimental.pallas import tpu_sc as plsc

NB_USERS = 10000
NB_MOVIES = 10000
K = 128
MAX_RATING = 5.0

NC = 2
NS = 16
NW = NC * NS

CH = 128
NCH = 124
N_PAD = NW * CH * NCH
NCH_P = NCH
PART = N_PAD
U_PAD = 10240
U_ROWS = U_PAD // NS
DUMMY_USER = NB_USERS

_ROW_CHUNK = 2048

_SC_PARAMS = pltpu.CompilerParams(use_tc_tiling_on_sc=False)
_SC_PARAMS_NL = pltpu.CompilerParams(use_tc_tiling_on_sc=False,
                                     needs_layout_passes=False)


def _mlp_body(g_ref, dr_ref, w1_ref, b1_ref, w2t_ref, b2_ref, v_ref):
    dr = dr_ref[:]
    s = jax.nn.sigmoid((dr * (1.0 / MAX_RATING)) * w1_ref[:] + b1_ref[:])
    t = g_ref[:] * s
    z = jnp.dot(t, w2t_ref[:], preferred_element_type=jnp.float32) + b2_ref[:]
    v_ref[:] = jax.nn.sigmoid(z)


def _mlp(g, dr, w1_row, b1_row, w2t, b2_row):
    n = g.shape[0]
    c = _ROW_CHUNK
    return pl.pallas_call(
        _mlp_body,
        grid=(n // c,),
        in_specs=[
            pl.BlockSpec((c, K), lambda i: (i, 0)),
            pl.BlockSpec((c, 1), lambda i: (i, 0)),
            pl.BlockSpec((1, K), lambda i: (0, 0)),
            pl.BlockSpec((1, K), lambda i: (0, 0)),
            pl.BlockSpec((K, K), lambda i: (0, 0)),
            pl.BlockSpec((1, K), lambda i: (0, 0)),
        ],
        out_specs=pl.BlockSpec((c, K), lambda i: (i, 0)),
        out_shape=jax.ShapeDtypeStruct((n, K), jnp.float32),
    )(g, dr, w1_row, b1_row, w2t, b2_row)


def _gather_body(tab_hbm, dm_hbm, g_out,
                 vb0, vb1, idx_b, sv0, sv1, sw0, sw1):
    c = lax.axis_index("c")
    s = lax.axis_index("s")
    w = s * NC + c
    rows_per_w = CH * NCH_P
    base0 = w * rows_per_w

    pltpu.sync_copy(dm_hbm.at[pl.ds(base0, rows_per_w)], idx_b)

    @pl.when(c == 1)
    def _():
        def add_off(i, carry):
            idx_b[pl.ds(i * 16, 16)] = idx_b[pl.ds(i * 16, 16)] + NB_MOVIES
            return carry
        lax.fori_loop(0, rows_per_w // 16, add_off, 0)

    GRP = 2 * CH
    ng = NCH_P // 2
    slots = ((vb0, sv0, sw0), (vb1, sv1, sw1))

    def fire(vb, sv, g):
        for j in range(2):
            idx = idx_b.at[pl.ds((g * 2 + j) * CH, CH)]
            pltpu.async_copy(tab_hbm.at[idx], vb.at[pl.ds(j * CH, CH)], sv)

    for b, (vb, sv, sw) in enumerate(slots):
        fire(vb, sv, b)

    def group_step(g, carry):
        for b, (vb, sv, sw) in enumerate(slots):
            gg = g * 2 + b
            for j in range(2):
                pltpu.make_async_copy(
                    tab_hbm.at[idx_b.at[pl.ds(0, CH)]],
                    vb.at[pl.ds(j * CH, CH)], sv).wait()
            pltpu.async_copy(vb, g_out.at[pl.ds(base0 + gg * GRP, GRP)], sw)

            @pl.when(gg + 2 < ng)
            def _():
                pltpu.make_async_copy(
                    vb, g_out.at[pl.ds(base0, GRP)], sw).wait()
                fire(vb, sv, gg + 2)
        return carry

    lax.fori_loop(0, ng // 2, group_step, 0)
    for b, (vb, sv, sw) in enumerate(slots):
        pltpu.make_async_copy(vb, g_out.at[pl.ds(base0, GRP)], sw).wait()


def _gather(tab, dm_part):
    mesh = plsc.VectorSubcoreMesh(core_axis_name="c", subcore_axis_name="s")
    return pl.kernel(
        _gather_body,
        out_type=jax.ShapeDtypeStruct((PART, K), jnp.float32),
        mesh=mesh,
        compiler_params=_SC_PARAMS,
        scratch_types=[
            pltpu.VMEM((2 * CH, K), jnp.float32),
            pltpu.VMEM((2 * CH, K), jnp.float32),
            pltpu.VMEM((CH * NCH_P,), jnp.int32),
            pltpu.SemaphoreType.DMA,
            pltpu.SemaphoreType.DMA,
            pltpu.SemaphoreType.DMA,
            pltpu.SemaphoreType.DMA,
        ],
    )(tab, dm_part)


def _scatter_body(v_hbm, du_hbm, acc_out,
                  vb0, vb1, ib0, ib1, zb, acc_sh, sv0, sv1, si0, si1):
    c = lax.axis_index("c")
    s = lax.axis_index("s")
    w = s * NC + c

    zeros16 = jnp.zeros((16,), jnp.float32)

    def fill_zb(i, carry):
        zb[i // 8, pl.ds((i % 8) * 16, 16)] = zeros16
        return carry
    lax.fori_loop(0, 8 * zb.shape[0], fill_zb, 0)

    row0 = s * U_ROWS
    for k in range(U_ROWS // zb.shape[0]):
        pltpu.sync_copy(zb, acc_sh.at[pl.ds(row0 + k * zb.shape[0], zb.shape[0])])
    plsc.subcore_barrier()

    slots = ((vb0, ib0, sv0, si0), (vb1, ib1, sv1, si1))

    for p in range(1):
        vbase = w * (CH * NCH_P)
        dbase = vbase
        for b, (vb, ib, sv, si) in enumerate(slots):
            pltpu.async_copy(v_hbm.at[pl.ds(vbase + b * CH, CH)], vb, sv)
            pltpu.async_copy(du_hbm.at[pl.ds(dbase + b * CH, CH)], ib, si)

        def chunk_pair(jj, carry):
            for b, (vb, ib, sv, si) in enumerate(slots):
                g = jj * 2 + b
                pltpu.make_async_copy(v_hbm.at[pl.ds(0, CH)], vb, sv).wait()
                pltpu.make_async_copy(du_hbm.at[pl.ds(0, CH)], ib, si).wait()
                pltpu.sync_copy(vb, acc_sh.at[ib], add=True)

                @pl.when(g + 2 < NCH_P)
                def _():
                    pltpu.async_copy(
                        v_hbm.at[pl.ds(vbase + (g + 2) * CH, CH)], vb, sv)
                    pltpu.async_copy(
                        du_hbm.at[pl.ds(dbase + (g + 2) * CH, CH)], ib, si)
            return carry

        lax.fori_loop(0, NCH_P // 2, chunk_pair, 0)

    plsc.subcore_barrier()
    pltpu.sync_copy(acc_sh.at[pl.ds(row0, U_ROWS)],
                    acc_out.at[c, pl.ds(row0, U_ROWS)])


def _scatter(v, du):
    mesh = plsc.VectorSubcoreMesh(core_axis_name="c", subcore_axis_name="s")
    return pl.kernel(
        _scatter_body,
        out_type=jax.ShapeDtypeStruct((NC, U_PAD, K), jnp.float32),
        mesh=mesh,
        compiler_params=_SC_PARAMS,
        scratch_types=[
            pltpu.VMEM((CH, K), jnp.float32),
            pltpu.VMEM((CH, K), jnp.float32),
            pltpu.VMEM((CH,), jnp.int32),
            pltpu.VMEM((CH,), jnp.int32),
            pltpu.VMEM((80, K), jnp.float32),
            pltpu.VMEM_SHARED((U_PAD, K), jnp.float32),
            pltpu.SemaphoreType.DMA,
            pltpu.SemaphoreType.DMA,
            pltpu.SemaphoreType.DMA,
            pltpu.SemaphoreType.DMA,
        ],
    )(v, du)


def _count_body(du_hbm, cnt_out, ib0, ib1, ones_b, zc, cnt_sh, si0, si1):
    c = lax.axis_index("c")
    s = lax.axis_index("s")
    w = s * NC + c

    zeros16 = jnp.zeros((16,), jnp.float32)
    ones16 = jnp.ones((16,), jnp.float32)

    def fill_zc(i, carry):
        zc[i, :] = zeros16
        return carry
    lax.fori_loop(0, zc.shape[0], fill_zc, 0)

    def fill_ones(i, carry):
        ones_b[i, :] = ones16
        return carry
    lax.fori_loop(0, ones_b.shape[0], fill_ones, 0)

    row0 = s * U_ROWS
    pltpu.sync_copy(zc, cnt_sh.at[pl.ds(row0, U_ROWS)])
    plsc.subcore_barrier()

    base0 = w * (CH * NCH)
    slots = ((ib0, si0), (ib1, si1))
    for b, (ib, si) in enumerate(slots):
        pltpu.async_copy(du_hbm.at[pl.ds(base0 + b * CH, CH)], ib, si)

    def chunk_pair(jj, carry):
        for b, (ib, si) in enumerate(slots):
            g = jj * 2 + b
            pltpu.make_async_copy(du_hbm.at[pl.ds(0, CH)], ib, si).wait()
            pltpu.sync_copy(ones_b, cnt_sh.at[ib], add=True)

            @pl.when(g + 2 < NCH)
            def _():
                nb = base0 + (g + 2) * CH
                pltpu.async_copy(du_hbm.at[pl.ds(nb, CH)], ib, si)
        return carry

    lax.fori_loop(0, NCH // 2, chunk_pair, 0)
    plsc.subcore_barrier()

    pltpu.sync_copy(cnt_sh.at[pl.ds(row0, U_ROWS)],
                    cnt_out.at[c, pl.ds(row0, U_ROWS)])


def _count(du):
    mesh = plsc.VectorSubcoreMesh(core_axis_name="c", subcore_axis_name="s")
    return pl.kernel(
        _count_body,
        out_type=jax.ShapeDtypeStruct((NC, U_PAD, 16), jnp.float32),
        mesh=mesh,
        compiler_params=_SC_PARAMS,
        scratch_types=[
            pltpu.VMEM((CH,), jnp.int32),
            pltpu.VMEM((CH,), jnp.int32),
            pltpu.VMEM((CH, 16), jnp.float32),
            pltpu.VMEM((U_ROWS, 16), jnp.float32),
            pltpu.VMEM_SHARED((U_PAD, 16), jnp.float32),
            pltpu.SemaphoreType.DMA,
            pltpu.SemaphoreType.DMA,
        ],
    )(du)


E_TILE = 1664
E_CHN = 13
E_PAD = NW * E_TILE


def _eval_body(uf_hbm, mov_hbm, xu_hbm, xm_hbm, out_hbm,
               ub0, mb0, ub1, mb1, xui, xmi, ob, su0, sm0, su1, sm1):
    c = lax.axis_index("c")
    s = lax.axis_index("s")
    w = s * NC + c
    base0 = w * E_TILE

    pltpu.sync_copy(xu_hbm.at[pl.ds(base0, E_TILE)], xui)
    pltpu.sync_copy(xm_hbm.at[pl.ds(base0, E_TILE)], xmi)

    slots = ((ub0, mb0, su0, sm0), (ub1, mb1, su1, sm1))

    def fire(b, ch):
        ub, mb, su, sm = slots[b]
        pltpu.async_copy(uf_hbm.at[xui.at[pl.ds(ch * CH, CH)]], ub, su)
        pltpu.async_copy(mov_hbm.at[xmi.at[pl.ds(ch * CH, CH)]], mb, sm)

    for b in range(2):
        fire(b, b)

    iota16 = jnp.arange(16, dtype=jnp.int32)

    def chunk_step(ch, carry):
        for b in range(2):
            @pl.when(ch * 2 + b < E_CHN)
            def _():
                cc = ch * 2 + b
                ub, mb, su, sm = slots[b]
                pltpu.make_async_copy(
                    uf_hbm.at[xui.at[pl.ds(0, CH)]], ub, su).wait()
                pltpu.make_async_copy(
                    mov_hbm.at[xmi.at[pl.ds(0, CH)]], mb, sm).wait()

                def group(g16, carry2):
                    rows16 = g16 * 16 + iota16

                    def jblk(jj, acc):
                        for j2 in range(32):
                            cols = jnp.full((16,), 0, jnp.int32) + (jj * 32 + j2)
                            u = plsc.load_gather(ub, [rows16, cols])
                            m = plsc.load_gather(mb, [rows16, cols])
                            acc = acc + u * m
                        return acc

                    acc = lax.fori_loop(0, 4, jblk,
                                        jnp.zeros((16,), jnp.float32))
                    ob[pl.ds(cc * CH + g16 * 16, 16)] = acc
                    return carry2

                lax.fori_loop(0, CH // 16, group, 0)

                @pl.when(cc + 2 < E_CHN)
                def _():
                    fire(b, cc + 2)
        return carry

    lax.fori_loop(0, (E_CHN + 1) // 2, chunk_step, 0)
    pltpu.sync_copy(ob, out_hbm.at[pl.ds(base0, E_TILE)])


def _eval_dots(uf, mov, xu, xm):
    mesh = plsc.VectorSubcoreMesh(core_axis_name="c", subcore_axis_name="s")
    return pl.kernel(
        _eval_body,
        out_type=jax.ShapeDtypeStruct((E_PAD,), jnp.float32),
        mesh=mesh,
        compiler_params=_SC_PARAMS_NL,
        scratch_types=[
            pltpu.VMEM((CH, K), jnp.float32),
            pltpu.VMEM((CH, K), jnp.float32),
            pltpu.VMEM((CH, K), jnp.float32),
            pltpu.VMEM((CH, K), jnp.float32),
            pltpu.VMEM((E_TILE,), jnp.int32),
            pltpu.VMEM((E_TILE,), jnp.int32),
            pltpu.VMEM((E_TILE,), jnp.float32),
            pltpu.SemaphoreType.DMA,
            pltpu.SemaphoreType.DMA,
            pltpu.SemaphoreType.DMA,
            pltpu.SemaphoreType.DMA,
        ],
    )(uf, mov, xu, xm)


def kernel(eval_xs, data_x, data_ratings, ignore_if_seen, movies_features,
           W1, b1, W2, b2, Wr, br):
    x_users = eval_xs[:, 0]
    x_movies = eval_xs[:, 1]
    n = data_x.shape[0]
    pad = N_PAD - n
    du = jnp.concatenate([data_x[:, 0],
                          jnp.full((pad,), DUMMY_USER, jnp.int32)])
    dm = jnp.concatenate([data_x[:, 1], jnp.zeros((pad,), jnp.int32)])
    dr = jnp.concatenate([data_ratings, jnp.zeros((pad,), jnp.float32)])

    w1r = W1.reshape(1, K)
    b1r = b1.reshape(1, K)
    w2t = W2.T
    b2r = b2.reshape(1, K)

    g = _gather(jnp.concatenate([movies_features, movies_features]), dm)
    v = _mlp(g, dr[:, None], w1r, b1r, w2t, b2r)

    acc_p = _scatter(v, du)
    cnt_p = _count(du)
    acc = acc_p[0] + acc_p[1]
    cnt = cnt_p[0, :, 0] + cnt_p[1, :, 0]

    scale = Wr[0, 0] * (MAX_RATING / K)
    uf_scaled = acc * (scale / cnt[:, None])

    epad = E_PAD - eval_xs.shape[0]
    xu = jnp.concatenate([x_users, jnp.zeros((epad,), jnp.int32)])
    xm = jnp.concatenate([x_movies, jnp.zeros((epad,), jnp.int32)])
    dots = _eval_dots(uf_scaled, movies_features, xu, xm)
    return dots[: eval_xs.shape[0]] + br[0] * MAX_RATING
